# Initial kernel scaffold; baseline (speedup 1.0000x reference)
import jax
import jax.numpy as jnp
from jax.experimental import pallas as pl


def kernel(x, W_gate, W_in, b_in, W_out, b_out):
    raise NotImplementedError("write your pallas kernel here")


# speedup vs baseline: 1.3482x; 1.3482x over previous
import functools

import jax
import jax.numpy as jnp
from jax.experimental import pallas as pl
from jax.experimental.pallas import tpu as pltpu

P, D, DMLP, E = 2048, 768, 3072, 8
TM = 512
NT = DMLP // TM


def _moe_body(x_ref, wg_ref, win_ref, bin_ref, wout_ref, bout_ref,
              out_ref, coef_ref):
    e = pl.program_id(0)
    t = pl.program_id(1)

    @pl.when((e == 0) & (t == 0))
    def _routing():
        g = jnp.dot(x_ref[...], wg_ref[...],
                    preferred_element_type=jnp.float32)
        lane = jax.lax.broadcasted_iota(jnp.int32, g.shape, 1)
        m1 = jnp.max(g, axis=1, keepdims=True)
        i1 = jnp.min(jnp.where(g == m1, lane, E), axis=1, keepdims=True)
        sel1 = lane == i1
        g2 = jnp.where(sel1, -jnp.inf, g)
        m2 = jnp.max(g2, axis=1, keepdims=True)
        i2 = jnp.min(jnp.where(g2 == m2, lane, E), axis=1, keepdims=True)
        sel2 = lane == i2
        r = jnp.exp(m2 - m1)
        w1 = 1.0 / (1.0 + r)
        w2 = r / (1.0 + r)
        contrib = jnp.where(sel1, w1, 0.0) + jnp.where(sel2, w2, 0.0)
        coefs = jnp.sum(contrib, axis=0, keepdims=True)
        coef_ref[...] = coefs
        out_ref[...] = jnp.broadcast_to(
            jnp.dot(coefs, bout_ref[...],
                    preferred_element_type=jnp.float32),
            out_ref.shape)

    lane_e = jax.lax.broadcasted_iota(jnp.int32, (1, E), 1)
    c = jnp.sum(jnp.where(lane_e == e, coef_ref[...], 0.0))
    pre = jnp.dot(x_ref[...], win_ref[0],
                  preferred_element_type=jnp.float32) + bin_ref[0]
    h = jnp.maximum(pre, 0.0) * c
    out_ref[...] += jnp.dot(h, wout_ref[0],
                            preferred_element_type=jnp.float32)


@jax.jit
def kernel(x, W_gate, W_in, b_in, W_out, b_out):
    B = x.shape[0]
    x2 = x.reshape(B * P, D)
    b_in3 = b_in.reshape(E, 1, DMLP)

    out = pl.pallas_call(
        _moe_body,
        grid=(E, NT),
        in_specs=[
            pl.BlockSpec((B * P, D), lambda e, t: (0, 0)),
            pl.BlockSpec((D, E), lambda e, t: (0, 0)),
            pl.BlockSpec((1, D, TM), lambda e, t: (e, 0, t)),
            pl.BlockSpec((1, 1, TM), lambda e, t: (e, 0, t)),
            pl.BlockSpec((1, TM, D), lambda e, t: (e, t, 0)),
            pl.BlockSpec((E, D), lambda e, t: (0, 0)),
        ],
        out_specs=pl.BlockSpec((B * P, D), lambda e, t: (0, 0)),
        out_shape=jax.ShapeDtypeStruct((B * P, D), jnp.float32),
        scratch_shapes=[pltpu.VMEM((1, E), jnp.float32)],
        compiler_params=pltpu.CompilerParams(
            dimension_semantics=("arbitrary", "arbitrary")),
    )(x2, W_gate, W_in, b_in3, W_out, b_out)
    return out.reshape(B, P, D)


# speedup vs baseline: 1.4386x; 1.0671x over previous
import functools

import jax
import jax.numpy as jnp
from jax.experimental import pallas as pl
from jax.experimental.pallas import tpu as pltpu

P, D, DMLP, E = 2048, 768, 3072, 8
TM = 1536
NT = DMLP // TM


def _moe_body(x_ref, wg_ref, win_ref, bin_ref, wout_ref, bout_ref,
              out_ref, coef_ref):
    e = pl.program_id(0)
    t = pl.program_id(1)

    @pl.when((e == 0) & (t == 0))
    def _routing():
        g = jnp.dot(x_ref[...], wg_ref[...],
                    preferred_element_type=jnp.float32)
        lane = jax.lax.broadcasted_iota(jnp.int32, g.shape, 1)
        m1 = jnp.max(g, axis=1, keepdims=True)
        i1 = jnp.min(jnp.where(g == m1, lane, E), axis=1, keepdims=True)
        sel1 = lane == i1
        g2 = jnp.where(sel1, -jnp.inf, g)
        m2 = jnp.max(g2, axis=1, keepdims=True)
        i2 = jnp.min(jnp.where(g2 == m2, lane, E), axis=1, keepdims=True)
        sel2 = lane == i2
        r = jnp.exp(m2 - m1)
        w1 = 1.0 / (1.0 + r)
        w2 = r / (1.0 + r)
        contrib = jnp.where(sel1, w1, 0.0) + jnp.where(sel2, w2, 0.0)
        coefs = jnp.sum(contrib, axis=0, keepdims=True)
        coef_ref[...] = coefs
        out_ref[...] = jnp.broadcast_to(
            jnp.dot(coefs, bout_ref[...],
                    preferred_element_type=jnp.float32),
            out_ref.shape)

    lane_e = jax.lax.broadcasted_iota(jnp.int32, (1, E), 1)
    c11 = jnp.sum(jnp.where(lane_e == e, coef_ref[...], 0.0),
                  axis=1, keepdims=True)
    pre = jnp.dot(x_ref[...], win_ref[0],
                  preferred_element_type=jnp.float32) + bin_ref[0]
    h = jnp.maximum(pre, 0.0)
    out_ref[...] += jnp.dot(h, wout_ref[0] * c11,
                            preferred_element_type=jnp.float32)


@jax.jit
def kernel(x, W_gate, W_in, b_in, W_out, b_out):
    B = x.shape[0]
    x2 = x.reshape(B * P, D)
    b_in3 = b_in.reshape(E, 1, DMLP)

    out = pl.pallas_call(
        _moe_body,
        grid=(E, NT),
        in_specs=[
            pl.BlockSpec((B * P, D), lambda e, t: (0, 0)),
            pl.BlockSpec((D, E), lambda e, t: (0, 0)),
            pl.BlockSpec((1, D, TM), lambda e, t: (e, 0, t)),
            pl.BlockSpec((1, 1, TM), lambda e, t: (e, 0, t)),
            pl.BlockSpec((1, TM, D), lambda e, t: (e, t, 0)),
            pl.BlockSpec((E, D), lambda e, t: (0, 0)),
        ],
        out_specs=pl.BlockSpec((B * P, D), lambda e, t: (0, 0)),
        out_shape=jax.ShapeDtypeStruct((B * P, D), jnp.float32),
        scratch_shapes=[pltpu.VMEM((1, E), jnp.float32)],
        compiler_params=pltpu.CompilerParams(
            dimension_semantics=("arbitrary", "arbitrary")),
    )(x2, W_gate, W_in, b_in3, W_out, b_out)
    return out.reshape(B, P, D)
